# Initial kernel scaffold; baseline (speedup 1.0000x reference)
import jax
import jax.numpy as jnp
from jax.experimental import pallas as pl


def kernel(logits, labels):
    raise NotImplementedError("write your pallas kernel here")


# speedup vs baseline: 35.5540x; 35.5540x over previous
import functools

import jax
import jax.numpy as jnp
from jax import lax
from jax.experimental import pallas as pl
from jax.experimental.pallas import tpu as pltpu
from jax.experimental.pallas import tpu_sc as plsc

NBINS = 2048
LANES = 16
NCORES = 2
NSUB = 16
NWORKERS = NCORES * NSUB
CHUNK = 16384


def _tc_softmax_err_body(lg_ref, lab_ref, z_ref):
    x = lg_ref[0]
    m = jnp.max(x, axis=0, keepdims=True)
    ex = jnp.exp(x - m)
    p = ex / jnp.sum(ex, axis=0, keepdims=True)
    lab = lab_ref[0]
    C, BH, W = x.shape
    cls = lax.broadcasted_iota(jnp.int32, (C, BH, W), 0)
    fg = lab[None, :, :] == cls
    z_ref[0] = jnp.where(fg, 3.0 - p, p)


def _tc_softmax_err(logits, labels):
    B, C, H, W = logits.shape
    BH = 64
    return pl.pallas_call(
        _tc_softmax_err_body,
        grid=(B, H // BH),
        in_specs=[
            pl.BlockSpec((1, C, BH, W), lambda b, h: (b, 0, h, 0)),
            pl.BlockSpec((1, BH, W), lambda b, h: (b, h, 0)),
        ],
        out_specs=pl.BlockSpec((1, C, BH, W), lambda b, h: (b, 0, h, 0)),
        out_shape=jax.ShapeDtypeStruct((B, C, H, W), jnp.float32),
    )(logits, labels)


def _sc_losses(z_flat, n_rows, n_pix):
    rows_padded = ((n_rows + NWORKERS - 1) // NWORKERS) * NWORKERS
    rows_per_w = rows_padded // NWORKERS
    n_chunks = n_pix // CHUNK
    vec_per_chunk = CHUNK // LANES
    hist_words = LANES * 2 * NBINS
    mesh = plsc.VectorSubcoreMesh(core_axis_name="c", subcore_axis_name="s")

    @functools.partial(
        pl.kernel,
        out_type=jax.ShapeDtypeStruct((rows_padded, LANES), jnp.float32),
        mesh=mesh,
        scratch_types=[
            pltpu.VMEM((hist_words,), jnp.float32),
            pltpu.VMEM((CHUNK,), jnp.float32),
            pltpu.VMEM((LANES,), jnp.float32),
        ],
        compiler_params=pltpu.CompilerParams(
            use_tc_tiling_on_sc=False, needs_layout_passes=False),
    )
    def body(z_hbm, out_hbm, hist, zbuf, outbuf):
        wid = lax.axis_index("s") * NCORES + lax.axis_index("c")
        lane = lax.iota(jnp.int32, LANES)
        lane_base = lane * (2 * NBINS)
        ones = jnp.ones((LANES,), jnp.float32)
        zerov = jnp.zeros((LANES,), jnp.float32)

        for t in range(rows_per_w):
            row = wid + NWORKERS * t

            @pl.when(row < n_rows)
            def _():
                def zero_body(i, carry):
                    hist[pl.ds(i * LANES, LANES)] = zerov
                    return carry
                lax.fori_loop(0, hist_words // LANES, zero_body, 0)

                def chunk_body(c, carry):
                    pltpu.sync_copy(
                        z_hbm.at[row, pl.ds(c * CHUNK, CHUNK)], zbuf)

                    def vec_body(v, carry2):
                        z = zbuf[pl.ds(v * LANES, LANES)]
                        fg = z >= 2.0
                        e = z - jnp.where(fg, 2.0, 0.0)
                        k = (e * float(NBINS)).astype(jnp.int32)
                        k = jnp.minimum(k, NBINS - 1)
                        idx = lane_base + jnp.where(fg, NBINS, 0) + k
                        plsc.addupdate_scatter(hist, [idx], ones)
                        return carry2
                    lax.fori_loop(0, vec_per_chunk, vec_body, 0)
                    return carry
                lax.fori_loop(0, n_chunks, chunk_body, 0)

                def red_body(j, gvec):
                    acc = hist[pl.ds(j * LANES, LANES)]
                    for l in range(1, LANES):
                        acc = acc + hist[pl.ds(l * 2 * NBINS + j * LANES,
                                               LANES)]
                    hist[pl.ds(j * LANES, LANES)] = acc
                    return gvec + jnp.where(j >= NBINS // LANES, acc, 0.0)
                gvec = lax.fori_loop(0, 2 * NBINS // LANES, red_body, zerov)
                G = jnp.sum(gvec)

                def jac_body(j, carry):
                    cN, cS, jvec = carry
                    jj = NBINS // LANES - 1 - j
                    h0 = hist[pl.ds(jj * LANES, LANES)]
                    h1 = hist[pl.ds(NBINS + jj * LANES, LANES)]
                    rn = lax.rev(h0 + h1, (0,))
                    rs = lax.rev(h1, (0,))
                    cn = plsc.cumsum(rn) + cN
                    cs = plsc.cumsum(rs) + cS
                    union = jnp.maximum(G + cn - cs, 1e-30)
                    J = jnp.where(cn > 0.5, 1.0 - (G - cs) / union, 0.0)
                    return jnp.max(cn), jnp.max(cs), jvec + J
                _, _, jvec = lax.fori_loop(
                    0, NBINS // LANES, jac_body,
                    (jnp.float32(0.0), jnp.float32(0.0), zerov))

                loss = (jnp.sum(jvec) - 0.5) * (1.0 / NBINS)
                outbuf[...] = zerov + loss
                pltpu.sync_copy(outbuf, out_hbm.at[row])

    return body(z_flat)


def kernel(logits, labels):
    B, C, H, W = logits.shape
    z = _tc_softmax_err(logits, labels)
    out = _sc_losses(z.reshape(B * C, H * W), B * C, H * W)
    return jnp.mean(out[: B * C, 0])


# speedup vs baseline: 49.5653x; 1.3941x over previous
import functools

import jax
import jax.numpy as jnp
from jax import lax
from jax.experimental import pallas as pl
from jax.experimental.pallas import tpu as pltpu
from jax.experimental.pallas import tpu_sc as plsc

NBINS = 2048
SCALE = float(NBINS) * (1.0 - 2.0 ** -20)
LANES = 16
NCORES = 2
NSUB = 16
NWORKERS = NCORES * NSUB
CHUNK = 16384


def _tc_softmax_err_body(lg_ref, lab_ref, z_ref):
    x = lg_ref[0]
    m = jnp.max(x, axis=0, keepdims=True)
    ex = jnp.exp(x - m)
    p = ex / jnp.sum(ex, axis=0, keepdims=True)
    lab = lab_ref[0]
    C, BH, W = x.shape
    cls = lax.broadcasted_iota(jnp.int32, (C, BH, W), 0)
    fg = lab[None, :, :] == cls
    z_ref[0] = jnp.where(fg, float(NBINS) + (1.0 - p) * SCALE, p * SCALE)


def _tc_softmax_err(logits, labels):
    B, C, H, W = logits.shape
    BH = 64
    return pl.pallas_call(
        _tc_softmax_err_body,
        grid=(B, H // BH),
        in_specs=[
            pl.BlockSpec((1, C, BH, W), lambda b, h: (b, 0, h, 0)),
            pl.BlockSpec((1, BH, W), lambda b, h: (b, h, 0)),
        ],
        out_specs=pl.BlockSpec((1, C, BH, W), lambda b, h: (b, 0, h, 0)),
        out_shape=jax.ShapeDtypeStruct((B, C, H, W), jnp.float32),
    )(logits, labels)


def _sc_losses(z_flat, n_rows, n_pix):
    rows_padded = ((n_rows + NWORKERS - 1) // NWORKERS) * NWORKERS
    rows_per_w = rows_padded // NWORKERS
    n_chunks = n_pix // CHUNK
    vec_per_chunk = CHUNK // LANES
    hist_words = LANES * 2 * NBINS
    mesh = plsc.VectorSubcoreMesh(core_axis_name="c", subcore_axis_name="s")

    @functools.partial(
        pl.kernel,
        out_type=jax.ShapeDtypeStruct((rows_padded, LANES), jnp.float32),
        mesh=mesh,
        scratch_types=[
            pltpu.VMEM((hist_words,), jnp.float32),
            pltpu.VMEM((CHUNK,), jnp.float32),
            pltpu.VMEM((CHUNK,), jnp.float32),
            pltpu.VMEM((LANES,), jnp.float32),
            pltpu.SemaphoreType.DMA,
            pltpu.SemaphoreType.DMA,
        ],
        compiler_params=pltpu.CompilerParams(
            use_tc_tiling_on_sc=False, needs_layout_passes=False),
    )
    def body(z_hbm, out_hbm, hist, zbuf0, zbuf1, outbuf, sem0, sem1):
        wid = lax.axis_index("s") * NCORES + lax.axis_index("c")
        lane = lax.iota(jnp.int32, LANES)
        lane_base = lane * (2 * NBINS)
        ones = jnp.ones((LANES,), jnp.float32)
        zerov = jnp.zeros((LANES,), jnp.float32)

        for t in range(rows_per_w):
            row = wid + NWORKERS * t

            @pl.when(row < n_rows)
            def _():
                def zero_body(i, carry):
                    hist[pl.ds(i * LANES, LANES)] = zerov
                    return carry
                lax.fori_loop(0, hist_words // LANES, zero_body, 0)

                UNROLL = 4

                def process(buf):
                    def vec_body(v, carry2):
                        for s in range(UNROLL):
                            z = buf[pl.ds((v * UNROLL + s) * LANES, LANES)]
                            idx = lane_base + z.astype(jnp.int32)
                            plsc.addupdate_scatter(hist, [idx], ones)
                        return carry2
                    lax.fori_loop(0, vec_per_chunk // UNROLL, vec_body, 0)

                def src(c):
                    return z_hbm.at[row, pl.ds(c * CHUNK, CHUNK)]

                pltpu.async_copy(src(0), zbuf0, sem0)

                def chunk_body(i, carry):
                    c0 = 2 * i
                    pltpu.make_async_copy(src(c0), zbuf0, sem0).wait()
                    pltpu.async_copy(src(c0 + 1), zbuf1, sem1)
                    process(zbuf0)
                    pltpu.make_async_copy(src(c0 + 1), zbuf1, sem1).wait()

                    @pl.when(c0 + 2 < n_chunks)
                    def _():
                        pltpu.async_copy(src(c0 + 2), zbuf0, sem0)
                    process(zbuf1)
                    return carry
                lax.fori_loop(0, n_chunks // 2, chunk_body, 0)

                def red_body(j, gvec):
                    acc = hist[pl.ds(j * LANES, LANES)]
                    for l in range(1, LANES):
                        acc = acc + hist[pl.ds(l * 2 * NBINS + j * LANES,
                                               LANES)]
                    hist[pl.ds(j * LANES, LANES)] = acc
                    return gvec + jnp.where(j >= NBINS // LANES, acc, 0.0)
                gvec = lax.fori_loop(0, 2 * NBINS // LANES, red_body, zerov)
                G = jnp.sum(gvec)

                def jac_body(j, carry):
                    cN, cS, jvec = carry
                    jj = NBINS // LANES - 1 - j
                    h0 = hist[pl.ds(jj * LANES, LANES)]
                    h1 = hist[pl.ds(NBINS + jj * LANES, LANES)]
                    rn = lax.rev(h0 + h1, (0,))
                    rs = lax.rev(h1, (0,))
                    cn = plsc.cumsum(rn) + cN
                    cs = plsc.cumsum(rs) + cS
                    union = jnp.maximum(G + cn - cs, 1e-30)
                    J = jnp.where(cn > 0.5, 1.0 - (G - cs) / union, 0.0)
                    return jnp.max(cn), jnp.max(cs), jvec + J
                _, _, jvec = lax.fori_loop(
                    0, NBINS // LANES, jac_body,
                    (jnp.float32(0.0), jnp.float32(0.0), zerov))

                loss = (jnp.sum(jvec) - 0.5) * (1.0 / NBINS)
                outbuf[...] = zerov + loss
                pltpu.sync_copy(outbuf, out_hbm.at[row])

    return body(z_flat)


def kernel(logits, labels):
    B, C, H, W = logits.shape
    z = _tc_softmax_err(logits, labels)
    out = _sc_losses(z.reshape(B * C, H * W), B * C, H * W)
    return jnp.mean(out[: B * C, 0])


# speedup vs baseline: 53.6600x; 1.0826x over previous
import functools

import jax
import jax.numpy as jnp
from jax import lax
from jax.experimental import pallas as pl
from jax.experimental.pallas import tpu as pltpu
from jax.experimental.pallas import tpu_sc as plsc

NBINS = 2048
SCALE = float(NBINS) * (1.0 - 2.0 ** -20)
LANES = 16
NCORES = 2
NSUB = 16
NWORKERS = NCORES * NSUB
CHUNK = 16384


def _tc_softmax_err_body(lg_ref, lab_ref, z_ref):
    x = lg_ref[0]
    m = jnp.max(x, axis=0, keepdims=True)
    ex = jnp.exp(x - m)
    p = ex / jnp.sum(ex, axis=0, keepdims=True)
    lab = lab_ref[0]
    C, BH, W = x.shape
    cls = lax.broadcasted_iota(jnp.int32, (C, BH, W), 0)
    fg = lab[None, :, :] == cls
    u = jnp.where(fg, float(NBINS) + jnp.floor((1.0 - p) * SCALE),
                  jnp.floor(p * SCALE))
    z_ref[0] = u * 16.0


def _tc_softmax_err(logits, labels):
    B, C, H, W = logits.shape
    BH = 64
    return pl.pallas_call(
        _tc_softmax_err_body,
        grid=(B, H // BH),
        in_specs=[
            pl.BlockSpec((1, C, BH, W), lambda b, h: (b, 0, h, 0)),
            pl.BlockSpec((1, BH, W), lambda b, h: (b, h, 0)),
        ],
        out_specs=pl.BlockSpec((1, C, BH, W), lambda b, h: (b, 0, h, 0)),
        out_shape=jax.ShapeDtypeStruct((B, C, H, W), jnp.float32),
    )(logits, labels)


def _sc_losses(z_flat, n_rows, n_pix):
    rows_padded = ((n_rows + NWORKERS - 1) // NWORKERS) * NWORKERS
    rows_per_w = rows_padded // NWORKERS
    n_chunks = n_pix // CHUNK
    vec_per_chunk = CHUNK // LANES
    hist_words = LANES * 2 * NBINS
    mesh = plsc.VectorSubcoreMesh(core_axis_name="c", subcore_axis_name="s")

    @functools.partial(
        pl.kernel,
        out_type=jax.ShapeDtypeStruct((rows_padded, LANES), jnp.float32),
        mesh=mesh,
        scratch_types=[
            pltpu.VMEM((hist_words,), jnp.float32),
            pltpu.VMEM((2 * NBINS,), jnp.float32),
            pltpu.VMEM((CHUNK,), jnp.float32),
            pltpu.VMEM((CHUNK,), jnp.float32),
            pltpu.VMEM((LANES,), jnp.float32),
            pltpu.SemaphoreType.DMA,
            pltpu.SemaphoreType.DMA,
        ],
        compiler_params=pltpu.CompilerParams(
            use_tc_tiling_on_sc=False, needs_layout_passes=False),
    )
    def body(z_hbm, out_hbm, hist, hred, zbuf0, zbuf1, outbuf, sem0, sem1):
        wid = lax.axis_index("s") * NCORES + lax.axis_index("c")
        lane = lax.iota(jnp.int32, LANES)
        ones = jnp.ones((LANES,), jnp.float32)
        zerov = jnp.zeros((LANES,), jnp.float32)

        def zero_body(i, carry):
            hist[pl.ds(i * LANES, LANES)] = zerov
            return carry
        lax.fori_loop(0, hist_words // LANES, zero_body, 0)

        for t in range(rows_per_w):
            row = wid + NWORKERS * t

            @pl.when(row < n_rows)
            def _():
                UNROLL = 8

                def process(buf):
                    def vec_body(v, carry2):
                        for s in range(UNROLL):
                            z = buf[pl.ds((v * UNROLL + s) * LANES, LANES)]
                            idx = z.astype(jnp.int32) + lane
                            plsc.addupdate_scatter(hist, [idx], ones)
                        return carry2
                    lax.fori_loop(0, vec_per_chunk // UNROLL, vec_body, 0)

                def src(c):
                    return z_hbm.at[row, pl.ds(c * CHUNK, CHUNK)]

                pltpu.async_copy(src(0), zbuf0, sem0)

                def chunk_body(i, carry):
                    c0 = 2 * i
                    pltpu.make_async_copy(src(c0), zbuf0, sem0).wait()
                    pltpu.async_copy(src(c0 + 1), zbuf1, sem1)
                    process(zbuf0)
                    pltpu.make_async_copy(src(c0 + 1), zbuf1, sem1).wait()

                    @pl.when(c0 + 2 < n_chunks)
                    def _():
                        pltpu.async_copy(src(c0 + 2), zbuf0, sem0)
                    process(zbuf1)
                    return carry
                lax.fori_loop(0, n_chunks // 2, chunk_body, 0)

                def red_body(tile, gvec):
                    acc = zerov
                    for j in range(LANES):
                        v = hist[pl.ds((tile * LANES + j) * LANES, LANES)]
                        hist[pl.ds((tile * LANES + j) * LANES, LANES)] = zerov
                        acc = jnp.where(lane == j, jnp.sum(v), acc)
                    hred[pl.ds(tile * LANES, LANES)] = acc
                    return gvec + jnp.where(tile >= NBINS // LANES, acc, 0.0)
                gvec = lax.fori_loop(0, 2 * NBINS // LANES, red_body, zerov)
                G = jnp.sum(gvec)

                def jac_body(j, carry):
                    cN, cS, jvec = carry
                    jj = NBINS // LANES - 1 - j
                    h0 = hred[pl.ds(jj * LANES, LANES)]
                    h1 = hred[pl.ds(NBINS + jj * LANES, LANES)]
                    rn = lax.rev(h0 + h1, (0,))
                    rs = lax.rev(h1, (0,))
                    cn = plsc.cumsum(rn) + cN
                    cs = plsc.cumsum(rs) + cS
                    union = jnp.maximum(G + cn - cs, 1e-30)
                    J = jnp.where(cn > 0.5, 1.0 - (G - cs) / union, 0.0)
                    return jnp.max(cn), jnp.max(cs), jvec + J
                _, _, jvec = lax.fori_loop(
                    0, NBINS // LANES, jac_body,
                    (jnp.float32(0.0), jnp.float32(0.0), zerov))

                loss = (jnp.sum(jvec) - 0.5) * (1.0 / NBINS)
                outbuf[...] = zerov + loss
                pltpu.sync_copy(outbuf, out_hbm.at[row])

    return body(z_flat)


def kernel(logits, labels):
    B, C, H, W = logits.shape
    z = _tc_softmax_err(logits, labels)
    out = _sc_losses(z.reshape(B * C, H * W), B * C, H * W)
    return jnp.mean(out[: B * C, 0])


# speedup vs baseline: 124.0978x; 2.3127x over previous
import functools

import jax
import jax.numpy as jnp
from jax import lax
from jax.experimental import pallas as pl
from jax.experimental.pallas import tpu as pltpu
from jax.experimental.pallas import tpu_sc as plsc

NBINS = 2048
SCALE = float(NBINS) * (1.0 - 2.0 ** -20)
LANES = 16
NCORES = 2
NSUB = 16
NWORKERS = NCORES * NSUB
CHUNK = 16384


def _tc_softmax_err_body(lg_ref, lab_ref, z_ref):
    x = lg_ref[0]
    m = jnp.max(x, axis=0, keepdims=True)
    ex = jnp.exp(x - m)
    p = ex / jnp.sum(ex, axis=0, keepdims=True)
    lab = lab_ref[0]
    C, BH, W = x.shape
    cls = lax.broadcasted_iota(jnp.int32, (C, BH, W), 0)
    fg = lab[None, :, :] == cls
    u = jnp.where(fg, float(NBINS) + jnp.floor((1.0 - p) * SCALE),
                  jnp.floor(p * SCALE))
    z_ref[0] = u * 16.0


def _tc_softmax_err(logits, labels):
    B, C, H, W = logits.shape
    BH = 64
    return pl.pallas_call(
        _tc_softmax_err_body,
        grid=(B, H // BH),
        in_specs=[
            pl.BlockSpec((1, C, BH, W), lambda b, h: (b, 0, h, 0)),
            pl.BlockSpec((1, BH, W), lambda b, h: (b, h, 0)),
        ],
        out_specs=pl.BlockSpec((1, C, BH, W), lambda b, h: (b, 0, h, 0)),
        out_shape=jax.ShapeDtypeStruct((B, C, H, W), jnp.float32),
    )(logits, labels)


def _sc_losses(z_flat, n_rows, n_pix):
    rows_padded = ((n_rows + NWORKERS - 1) // NWORKERS) * NWORKERS
    rows_per_w = rows_padded // NWORKERS
    n_chunks = n_pix // CHUNK
    vec_per_chunk = CHUNK // LANES
    hist_words = LANES * 2 * NBINS
    mesh = plsc.VectorSubcoreMesh(core_axis_name="c", subcore_axis_name="s")

    @functools.partial(
        pl.kernel,
        out_type=jax.ShapeDtypeStruct((rows_padded, LANES), jnp.float32),
        mesh=mesh,
        scratch_types=[
            pltpu.VMEM((hist_words,), jnp.float32),
            pltpu.VMEM((2 * NBINS,), jnp.float32),
            pltpu.VMEM((CHUNK,), jnp.float32),
            pltpu.VMEM((CHUNK,), jnp.float32),
            pltpu.VMEM((LANES,), jnp.float32),
            pltpu.SemaphoreType.DMA,
            pltpu.SemaphoreType.DMA,
        ],
        compiler_params=pltpu.CompilerParams(
            use_tc_tiling_on_sc=False, needs_layout_passes=False),
    )
    def body(z_hbm, out_hbm, hist, hred, zbuf0, zbuf1, outbuf, sem0, sem1):
        wid = lax.axis_index("s") * NCORES + lax.axis_index("c")
        lane = lax.iota(jnp.int32, LANES)
        ones = jnp.ones((LANES,), jnp.float32)
        zerov = jnp.zeros((LANES,), jnp.float32)

        def zero_body(i, carry):
            hist[pl.ds(i * LANES, LANES)] = zerov
            return carry
        lax.fori_loop(0, hist_words // LANES, zero_body, 0)

        for t in range(rows_per_w):
            row = wid + NWORKERS * t

            @pl.when(row < n_rows)
            def _():
                def process(buf):
                    @plsc.parallel_loop(0, vec_per_chunk, unroll=8)
                    def _(v):
                        z = buf[pl.ds(v * LANES, LANES)]
                        idx = z.astype(jnp.int32) + lane
                        plsc.addupdate_scatter(hist, [idx], ones)

                def src(c):
                    return z_hbm.at[row, pl.ds(c * CHUNK, CHUNK)]

                pltpu.async_copy(src(0), zbuf0, sem0)

                def chunk_body(i, carry):
                    c0 = 2 * i
                    pltpu.make_async_copy(src(c0), zbuf0, sem0).wait()
                    pltpu.async_copy(src(c0 + 1), zbuf1, sem1)
                    process(zbuf0)
                    pltpu.make_async_copy(src(c0 + 1), zbuf1, sem1).wait()

                    @pl.when(c0 + 2 < n_chunks)
                    def _():
                        pltpu.async_copy(src(c0 + 2), zbuf0, sem0)
                    process(zbuf1)
                    return carry
                lax.fori_loop(0, n_chunks // 2, chunk_body, 0)

                def red_body(tile, gvec):
                    acc = zerov
                    for j in range(LANES):
                        v = hist[pl.ds((tile * LANES + j) * LANES, LANES)]
                        hist[pl.ds((tile * LANES + j) * LANES, LANES)] = zerov
                        acc = jnp.where(lane == j, jnp.sum(v), acc)
                    hred[pl.ds(tile * LANES, LANES)] = acc
                    return gvec + jnp.where(tile >= NBINS // LANES, acc, 0.0)
                gvec = lax.fori_loop(0, 2 * NBINS // LANES, red_body, zerov)
                G = jnp.sum(gvec)

                def jac_body(j, carry):
                    cN, cS, jvec = carry
                    jj = NBINS // LANES - 1 - j
                    h0 = hred[pl.ds(jj * LANES, LANES)]
                    h1 = hred[pl.ds(NBINS + jj * LANES, LANES)]
                    rn = lax.rev(h0 + h1, (0,))
                    rs = lax.rev(h1, (0,))
                    cn = plsc.cumsum(rn) + cN
                    cs = plsc.cumsum(rs) + cS
                    union = jnp.maximum(G + cn - cs, 1e-30)
                    J = jnp.where(cn > 0.5, 1.0 - (G - cs) / union, 0.0)
                    return jnp.max(cn), jnp.max(cs), jvec + J
                _, _, jvec = lax.fori_loop(
                    0, NBINS // LANES, jac_body,
                    (jnp.float32(0.0), jnp.float32(0.0), zerov))

                loss = (jnp.sum(jvec) - 0.5) * (1.0 / NBINS)
                outbuf[...] = zerov + loss
                pltpu.sync_copy(outbuf, out_hbm.at[row])

    return body(z_flat)


def kernel(logits, labels):
    B, C, H, W = logits.shape
    z = _tc_softmax_err(logits, labels)
    out = _sc_losses(z.reshape(B * C, H * W), B * C, H * W)
    return jnp.mean(out[: B * C, 0])
